# baseline (speedup 1.0000x reference)
import functools

import jax
import jax.numpy as jnp
from jax import lax
from jax.experimental import pallas as pl
from jax.experimental.pallas import tpu as pltpu
from jax.experimental.pallas import tpu_sc as plsc

B, F, L, V, D = 16384, 26, 20, 1000000, 16
H = F * D
NC, NS = 2, 16
NW = NC * NS

VB = 2048
NFULL = V // VB
VTAIL = V - NFULL * VB
NCHUNK = NFULL + 1


def _transpose_sc(tbl_t, tbl_tail):
    mesh = plsc.VectorSubcoreMesh(core_axis_name="c", subcore_axis_name="s")

    @functools.partial(
        pl.kernel,
        out_type=jax.ShapeDtypeStruct((V * D // 128, 128), jnp.float32),
        mesh=mesh,
        scratch_types=[
            pltpu.VMEM((D, VB), jnp.float32),
            pltpu.VMEM((VB // 8, 128), jnp.float32),
        ],
        compiler_params=pltpu.CompilerParams(needs_layout_passes=False),
    )
    def tr_kernel(t_hbm, tail_hbm, out_hbm, in_v, out_v):
        wid = lax.axis_index("s") * NC + lax.axis_index("c")
        row_iota = lax.iota(jnp.int32, 16)

        def pack_cols(nv):
            @pl.loop(0, nv // 8)
            def _row(r):
                for j in range(8):
                    col = jnp.full((16,), r * 8 + j, jnp.int32)
                    out_v[r, pl.ds(j * D, D)] = plsc.load_gather(
                        in_v, [row_iota, col])

        @pl.loop(0, (NCHUNK + NW - 1) // NW)
        def _k(k):
            c = wid + k * NW

            @pl.when(c < NFULL)
            def _():
                pltpu.sync_copy(t_hbm.at[:, pl.ds(c * VB, VB)], in_v)
                pack_cols(VB)
                pltpu.sync_copy(out_v,
                                out_hbm.at[pl.ds(c * (VB // 8), VB // 8)])

            @pl.when(c == NFULL)
            def _():
                pltpu.sync_copy(tail_hbm, in_v.at[:, pl.ds(0, 640)])
                pack_cols(640)
                pltpu.sync_copy(out_v.at[pl.ds(0, VTAIL // 8)],
                                out_hbm.at[pl.ds(NFULL * (VB // 8),
                                                 VTAIL // 8)])

    return tr_kernel(tbl_t, tbl_tail)


BBLK = 128
NBLK = B // BBLK
ITEMS = F * NBLK
IPW = ITEMS // NW


def _pool_sc(x3, table):
    mesh = plsc.VectorSubcoreMesh(core_axis_name="c", subcore_axis_name="s")

    @functools.partial(
        pl.kernel,
        out_type=jax.ShapeDtypeStruct((B, F, D), jnp.float32),
        mesh=mesh,
        scratch_types=[
            pltpu.VMEM((2, L, BBLK), jnp.int32),
            pltpu.VMEM((L * BBLK, D), jnp.float32),
            pltpu.VMEM((L * BBLK, D), jnp.float32),
            pltpu.VMEM((BBLK, D), jnp.float32),
            pltpu.SemaphoreType.DMA,
            pltpu.SemaphoreType.DMA,
            pltpu.SemaphoreType.DMA,
            pltpu.SemaphoreType.DMA,
        ],
        compiler_params=pltpu.CompilerParams(use_tc_tiling_on_sc=False),
    )
    def pool_kernel(x_hbm, tbl_hbm, out_hbm, idx_v, rows0_v, rows1_v,
                    pooled_v, si0, si1, sg0, sg1):
        wid = lax.axis_index("s") * NC + lax.axis_index("c")
        rows = (rows0_v, rows1_v)
        isem = (si0, si1)
        gsem = (sg0, sg1)

        def split(t):
            item = wid * IPW + t
            f = lax.shift_right_logical(item, 7)
            b0 = lax.mul(lax.rem(item, NBLK), BBLK)
            return f, b0

        def fire_idx(slot, t):
            f, b0 = split(t)
            pltpu.async_copy(x_hbm.at[f, :, pl.ds(b0, BBLK)], idx_v.at[slot],
                             isem[slot])

        def wait_idx(slot):
            pltpu.make_async_copy(x_hbm.at[0, :, pl.ds(0, BBLK)],
                                  idx_v.at[slot], isem[slot]).wait()

        def fire_gather(slot):
            wait_idx(slot)
            for l in range(L):
                pltpu.async_copy(
                    tbl_hbm.at[idx_v.at[slot].at[l]],
                    rows[slot].at[pl.ds(l * BBLK, BBLK)],
                    gsem[slot],
                )

        def wait_gathers(slot):
            for l in range(L):
                pltpu.make_async_copy(
                    tbl_hbm.at[idx_v.at[slot].at[l]],
                    rows[slot].at[pl.ds(l * BBLK, BBLK)],
                    gsem[slot],
                ).wait()

        def acc_out(slot, t):
            @pl.loop(0, BBLK)
            def _seg(bb):
                vals = [rows[slot][l * BBLK + bb] for l in range(L)]
                while len(vals) > 1:
                    nxt = [vals[i] + vals[i + 1]
                           for i in range(0, len(vals) - 1, 2)]
                    if len(vals) % 2:
                        nxt.append(vals[-1])
                    vals = nxt
                pooled_v[bb] = vals[0]

            f, b0 = split(t)
            pltpu.sync_copy(pooled_v, out_hbm.at[pl.ds(b0, BBLK), f])

        fire_idx(0, 0)
        fire_idx(1, 1)
        fire_gather(0)

        @pl.loop(0, IPW // 2)
        def _pair(tt):
            t0 = tt * 2
            fire_gather(1)
            wait_gathers(0)

            @pl.when(t0 + 2 < IPW)
            def _():
                fire_idx(0, t0 + 2)

            acc_out(0, t0)

            @pl.when(t0 + 2 < IPW)
            def _():
                fire_gather(0)

            wait_gathers(1)

            @pl.when(t0 + 3 < IPW)
            def _():
                fire_idx(1, t0 + 3)

            acc_out(1, t0 + 1)

    return pool_kernel(x3, table)


BB = 512


def _mlp_tc(sparse, gamma, beta, W1, b1, W2, b2, W3, b3):
    def body(p_ref, g_ref, be_ref, w1_ref, b1_ref, w2_ref, b2_ref, w3_ref,
             b3_ref, o_ref):
        sp = p_ref[...]
        mu = jnp.mean(sp, axis=-1, keepdims=True)
        var = jnp.mean((sp - mu) ** 2, axis=-1, keepdims=True)
        h = (sp - mu) / jnp.sqrt(var + 1e-5) * g_ref[...] + be_ref[...]
        h = jnp.maximum(
            jnp.dot(h.astype(jnp.bfloat16), w1_ref[...].astype(jnp.bfloat16),
                    preferred_element_type=jnp.float32)
            + b1_ref[...], 0.0)
        h = jnp.maximum(
            jnp.dot(h.astype(jnp.bfloat16), w2_ref[...].astype(jnp.bfloat16),
                    preferred_element_type=jnp.float32)
            + b2_ref[...], 0.0)
        o = jnp.dot(h, w3_ref[...], preferred_element_type=jnp.float32) + b3_ref[...]
        o_ref[...] = jax.nn.sigmoid(o)

    return pl.pallas_call(
        body,
        grid=(B // BB,),
        in_specs=[
            pl.BlockSpec((BB, H), lambda i: (i, 0)),
            pl.BlockSpec((H,), lambda i: (0,)),
            pl.BlockSpec((H,), lambda i: (0,)),
            pl.BlockSpec((H, 1024), lambda i: (0, 0)),
            pl.BlockSpec((1024,), lambda i: (0,)),
            pl.BlockSpec((1024, 512), lambda i: (0, 0)),
            pl.BlockSpec((512,), lambda i: (0,)),
            pl.BlockSpec((512, 1), lambda i: (0, 0)),
            pl.BlockSpec((1,), lambda i: (0,)),
        ],
        out_specs=pl.BlockSpec((BB, 1), lambda i: (i, 0)),
        out_shape=jax.ShapeDtypeStruct((B, 1), jnp.float32),
    )(sparse, gamma, beta, W1, b1, W2, b2, W3, b3)


def kernel(x, table, gamma, beta, W1, b1, W2, b2, W3, b3):
    x3 = x.transpose(1, 2, 0)
    tbl_t = table.T
    tail = jnp.pad(tbl_t[:, NFULL * VB:], ((0, 0), (0, 640 - VTAIL)))
    tbl_lin = _transpose_sc(tbl_t, tail).reshape(V, D)
    pooled = _pool_sc(x3, tbl_lin)
    sparse = pooled.reshape(B, H)
    return _mlp_tc(sparse, gamma, beta, W1, b1, W2, b2, W3, b3)


# speedup vs baseline: 1.1703x; 1.1703x over previous
import functools

import jax
import jax.numpy as jnp
from jax import lax
from jax.experimental import pallas as pl
from jax.experimental.pallas import tpu as pltpu
from jax.experimental.pallas import tpu_sc as plsc

B, F, L, V, D = 16384, 26, 20, 1000000, 16
H = F * D
NC, NS = 2, 16
NW = NC * NS

VB = 2048
NFULL = V // VB
VTAIL = V - NFULL * VB
NCHUNK = NFULL + 1


def _transpose_sc(tbl_t, tbl_tail):
    mesh = plsc.VectorSubcoreMesh(core_axis_name="c", subcore_axis_name="s")

    @functools.partial(
        pl.kernel,
        out_type=jax.ShapeDtypeStruct((V * D // 128, 128), jnp.float32),
        mesh=mesh,
        scratch_types=[
            pltpu.VMEM((D, VB), jnp.float32),
            pltpu.VMEM((VB // 8, 128), jnp.float32),
        ],
        compiler_params=pltpu.CompilerParams(needs_layout_passes=False),
    )
    def tr_kernel(t_hbm, tail_hbm, out_hbm, in_v, out_v):
        wid = lax.axis_index("s") * NC + lax.axis_index("c")
        row_iota = lax.iota(jnp.int32, 16)

        def pack_cols(nv):
            @pl.loop(0, nv // 16, step=1)
            def _row(rr):
                r = rr * 2
                vals = [plsc.load_gather(
                            in_v, [row_iota,
                                   jnp.full((16,), r * 8 + j, jnp.int32)])
                        for j in range(16)]
                for j in range(8):
                    out_v[r, pl.ds(j * D, D)] = vals[j]
                for j in range(8):
                    out_v[r + 1, pl.ds(j * D, D)] = vals[8 + j]

        @pl.loop(0, (NCHUNK + NW - 1) // NW)
        def _k(k):
            c = wid + k * NW

            @pl.when(c < NFULL)
            def _():
                pltpu.sync_copy(t_hbm.at[:, pl.ds(c * VB, VB)], in_v)
                pack_cols(VB)
                pltpu.sync_copy(out_v,
                                out_hbm.at[pl.ds(c * (VB // 8), VB // 8)])

            @pl.when(c == NFULL)
            def _():
                pltpu.sync_copy(tail_hbm, in_v.at[:, pl.ds(0, 640)])
                pack_cols(640)
                pltpu.sync_copy(out_v.at[pl.ds(0, VTAIL // 8)],
                                out_hbm.at[pl.ds(NFULL * (VB // 8),
                                                 VTAIL // 8)])

    return tr_kernel(tbl_t, tbl_tail)


BBLK = 128
NBLK = B // BBLK
ITEMS = F * NBLK
IPW = ITEMS // NW


def _pool_sc(x3, table):
    mesh = plsc.VectorSubcoreMesh(core_axis_name="c", subcore_axis_name="s")

    @functools.partial(
        pl.kernel,
        out_type=jax.ShapeDtypeStruct((B, F, D), jnp.float32),
        mesh=mesh,
        scratch_types=[
            pltpu.VMEM((2, L, BBLK), jnp.int32),
            pltpu.VMEM((L * BBLK, D), jnp.float32),
            pltpu.VMEM((L * BBLK, D), jnp.float32),
            pltpu.VMEM((BBLK, D), jnp.float32),
            pltpu.SemaphoreType.DMA,
            pltpu.SemaphoreType.DMA,
            pltpu.SemaphoreType.DMA,
            pltpu.SemaphoreType.DMA,
        ],
        compiler_params=pltpu.CompilerParams(use_tc_tiling_on_sc=False),
    )
    def pool_kernel(x_hbm, tbl_hbm, out_hbm, idx_v, rows0_v, rows1_v,
                    pooled_v, si0, si1, sg0, sg1):
        wid = lax.axis_index("s") * NC + lax.axis_index("c")
        rows = (rows0_v, rows1_v)
        isem = (si0, si1)
        gsem = (sg0, sg1)

        def split(t):
            item = wid * IPW + t
            f = lax.shift_right_logical(item, 7)
            b0 = lax.mul(lax.rem(item, NBLK), BBLK)
            return f, b0

        def fire_idx(slot, t):
            f, b0 = split(t)
            pltpu.async_copy(x_hbm.at[f, :, pl.ds(b0, BBLK)], idx_v.at[slot],
                             isem[slot])

        def wait_idx(slot):
            pltpu.make_async_copy(x_hbm.at[0, :, pl.ds(0, BBLK)],
                                  idx_v.at[slot], isem[slot]).wait()

        def fire_gather(slot):
            wait_idx(slot)
            for l in range(L):
                pltpu.async_copy(
                    tbl_hbm.at[idx_v.at[slot].at[l]],
                    rows[slot].at[pl.ds(l * BBLK, BBLK)],
                    gsem[slot],
                )

        def wait_gathers(slot):
            for l in range(L):
                pltpu.make_async_copy(
                    tbl_hbm.at[idx_v.at[slot].at[l]],
                    rows[slot].at[pl.ds(l * BBLK, BBLK)],
                    gsem[slot],
                ).wait()

        def acc_out(slot, t):
            @pl.loop(0, BBLK)
            def _seg(bb):
                vals = [rows[slot][l * BBLK + bb] for l in range(L)]
                while len(vals) > 1:
                    nxt = [vals[i] + vals[i + 1]
                           for i in range(0, len(vals) - 1, 2)]
                    if len(vals) % 2:
                        nxt.append(vals[-1])
                    vals = nxt
                pooled_v[bb] = vals[0]

            f, b0 = split(t)
            pltpu.sync_copy(pooled_v, out_hbm.at[pl.ds(b0, BBLK), f])

        fire_idx(0, 0)
        fire_idx(1, 1)
        fire_gather(0)

        @pl.loop(0, IPW // 2)
        def _pair(tt):
            t0 = tt * 2
            fire_gather(1)
            wait_gathers(0)

            @pl.when(t0 + 2 < IPW)
            def _():
                fire_idx(0, t0 + 2)

            acc_out(0, t0)

            @pl.when(t0 + 2 < IPW)
            def _():
                fire_gather(0)

            wait_gathers(1)

            @pl.when(t0 + 3 < IPW)
            def _():
                fire_idx(1, t0 + 3)

            acc_out(1, t0 + 1)

    return pool_kernel(x3, table)


BB = 512


def _mlp_tc(sparse, gamma, beta, W1, b1, W2, b2, W3, b3):
    def body(p_ref, g_ref, be_ref, w1_ref, b1_ref, w2_ref, b2_ref, w3_ref,
             b3_ref, o_ref):
        sp = p_ref[...]
        mu = jnp.mean(sp, axis=-1, keepdims=True)
        var = jnp.mean((sp - mu) ** 2, axis=-1, keepdims=True)
        h = (sp - mu) / jnp.sqrt(var + 1e-5) * g_ref[...] + be_ref[...]
        h = jnp.maximum(
            jnp.dot(h.astype(jnp.bfloat16), w1_ref[...].astype(jnp.bfloat16),
                    preferred_element_type=jnp.float32)
            + b1_ref[...], 0.0)
        h = jnp.maximum(
            jnp.dot(h.astype(jnp.bfloat16), w2_ref[...].astype(jnp.bfloat16),
                    preferred_element_type=jnp.float32)
            + b2_ref[...], 0.0)
        o = jnp.dot(h, w3_ref[...], preferred_element_type=jnp.float32) + b3_ref[...]
        o_ref[...] = jax.nn.sigmoid(o)

    return pl.pallas_call(
        body,
        grid=(B // BB,),
        in_specs=[
            pl.BlockSpec((BB, H), lambda i: (i, 0)),
            pl.BlockSpec((H,), lambda i: (0,)),
            pl.BlockSpec((H,), lambda i: (0,)),
            pl.BlockSpec((H, 1024), lambda i: (0, 0)),
            pl.BlockSpec((1024,), lambda i: (0,)),
            pl.BlockSpec((1024, 512), lambda i: (0, 0)),
            pl.BlockSpec((512,), lambda i: (0,)),
            pl.BlockSpec((512, 1), lambda i: (0, 0)),
            pl.BlockSpec((1,), lambda i: (0,)),
        ],
        out_specs=pl.BlockSpec((BB, 1), lambda i: (i, 0)),
        out_shape=jax.ShapeDtypeStruct((B, 1), jnp.float32),
    )(sparse, gamma, beta, W1, b1, W2, b2, W3, b3)


def kernel(x, table, gamma, beta, W1, b1, W2, b2, W3, b3):
    x3 = x.transpose(1, 2, 0)
    tbl_t = table.T
    tail = jnp.pad(tbl_t[:, NFULL * VB:], ((0, 0), (0, 640 - VTAIL)))
    tbl_lin = _transpose_sc(tbl_t, tail).reshape(V, D)
    pooled = _pool_sc(x3, tbl_lin)
    sparse = pooled.reshape(B, H)
    return _mlp_tc(sparse, gamma, beta, W1, b1, W2, b2, W3, b3)


# speedup vs baseline: 1.3167x; 1.1251x over previous
import functools

import jax
import jax.numpy as jnp
from jax import lax
from jax.experimental import pallas as pl
from jax.experimental.pallas import tpu as pltpu
from jax.experimental.pallas import tpu_sc as plsc

B, F, L, V, D = 16384, 26, 20, 1000000, 16
H = F * D
NC, NS = 2, 16
NW = NC * NS

VB = 2048
NFULL = V // VB
VTAIL = V - NFULL * VB
NCHUNK = NFULL + 1


def _transpose_sc(tbl_t, tbl_tail):
    mesh = plsc.VectorSubcoreMesh(core_axis_name="c", subcore_axis_name="s")

    @functools.partial(
        pl.kernel,
        out_type=jax.ShapeDtypeStruct((V * D // 128, 128), jnp.float32),
        mesh=mesh,
        scratch_types=[
            pltpu.VMEM((D, VB), jnp.float32),
            pltpu.VMEM((VB // 8, 128), jnp.float32),
        ],
        compiler_params=pltpu.CompilerParams(needs_layout_passes=False),
    )
    def tr_kernel(t_hbm, tail_hbm, out_hbm, in_v, out_v):
        wid = lax.axis_index("s") * NC + lax.axis_index("c")
        row_iota = lax.iota(jnp.int32, 16)

        def pack_cols(nv):
            @pl.loop(0, nv // 16, step=1)
            def _row(rr):
                r = rr * 2
                vals = [plsc.load_gather(
                            in_v, [row_iota,
                                   jnp.full((16,), r * 8 + j, jnp.int32)])
                        for j in range(16)]
                for j in range(8):
                    out_v[r, pl.ds(j * D, D)] = vals[j]
                for j in range(8):
                    out_v[r + 1, pl.ds(j * D, D)] = vals[8 + j]

        @pl.loop(0, (NCHUNK + NW - 1) // NW)
        def _k(k):
            c = wid + k * NW

            @pl.when(c < NFULL)
            def _():
                pltpu.sync_copy(t_hbm.at[:, pl.ds(c * VB, VB)], in_v)
                pack_cols(VB)
                pltpu.sync_copy(out_v,
                                out_hbm.at[pl.ds(c * (VB // 8), VB // 8)])

            @pl.when(c == NFULL)
            def _():
                pltpu.sync_copy(tail_hbm, in_v.at[:, pl.ds(0, 640)])
                pack_cols(640)
                pltpu.sync_copy(out_v.at[pl.ds(0, VTAIL // 8)],
                                out_hbm.at[pl.ds(NFULL * (VB // 8),
                                                 VTAIL // 8)])

    return tr_kernel(tbl_t, tbl_tail)


BBLK = 128
NBLK = B // BBLK
ITEMS = F * NBLK
IPW = ITEMS // NW


def _pool_sc(x3, table, nblk, base_blk):
    mesh = plsc.VectorSubcoreMesh(core_axis_name="c", subcore_axis_name="s")
    items = F * nblk
    ipw = items // NW
    shift = nblk.bit_length() - 1

    @functools.partial(
        pl.kernel,
        out_type=jax.ShapeDtypeStruct((nblk * BBLK, F, D), jnp.float32),
        mesh=mesh,
        scratch_types=[
            pltpu.VMEM((2, L, BBLK), jnp.int32),
            pltpu.VMEM((L * BBLK, D), jnp.float32),
            pltpu.VMEM((L * BBLK, D), jnp.float32),
            pltpu.VMEM((BBLK, D), jnp.float32),
            pltpu.SemaphoreType.DMA,
            pltpu.SemaphoreType.DMA,
            pltpu.SemaphoreType.DMA,
            pltpu.SemaphoreType.DMA,
        ],
        compiler_params=pltpu.CompilerParams(use_tc_tiling_on_sc=False),
    )
    def pool_kernel(x_hbm, tbl_hbm, out_hbm, idx_v, rows0_v, rows1_v,
                    pooled_v, si0, si1, sg0, sg1):
        wid = lax.axis_index("s") * NC + lax.axis_index("c")
        rows = (rows0_v, rows1_v)
        isem = (si0, si1)
        gsem = (sg0, sg1)

        def split(t):
            item = wid * ipw + t
            f = lax.shift_right_logical(item, shift)
            b0 = lax.mul(lax.rem(item, nblk), BBLK)
            return f, b0

        def fire_idx(slot, t):
            f, b0 = split(t)
            pltpu.async_copy(
                x_hbm.at[f, :, pl.ds(b0 + base_blk * BBLK, BBLK)],
                idx_v.at[slot], isem[slot])

        def wait_idx(slot):
            pltpu.make_async_copy(x_hbm.at[0, :, pl.ds(0, BBLK)],
                                  idx_v.at[slot], isem[slot]).wait()

        def fire_gather(slot):
            wait_idx(slot)
            for l in range(L):
                pltpu.async_copy(
                    tbl_hbm.at[idx_v.at[slot].at[l]],
                    rows[slot].at[pl.ds(l * BBLK, BBLK)],
                    gsem[slot],
                )

        def wait_gathers(slot):
            for l in range(L):
                pltpu.make_async_copy(
                    tbl_hbm.at[idx_v.at[slot].at[l]],
                    rows[slot].at[pl.ds(l * BBLK, BBLK)],
                    gsem[slot],
                ).wait()

        def acc_out(slot, t):
            @pl.loop(0, BBLK)
            def _seg(bb):
                vals = [rows[slot][l * BBLK + bb] for l in range(L)]
                while len(vals) > 1:
                    nxt = [vals[i] + vals[i + 1]
                           for i in range(0, len(vals) - 1, 2)]
                    if len(vals) % 2:
                        nxt.append(vals[-1])
                    vals = nxt
                pooled_v[bb] = vals[0]

            f, b0 = split(t)
            pltpu.sync_copy(pooled_v, out_hbm.at[pl.ds(b0, BBLK), f])

        fire_idx(0, 0)
        fire_idx(1, 1)
        fire_gather(0)

        @pl.loop(0, ipw // 2)
        def _pair(tt):
            t0 = tt * 2
            fire_gather(1)
            wait_gathers(0)

            @pl.when(t0 + 2 < ipw)
            def _():
                fire_idx(0, t0 + 2)

            acc_out(0, t0)

            @pl.when(t0 + 2 < ipw)
            def _():
                fire_gather(0)

            wait_gathers(1)

            @pl.when(t0 + 3 < ipw)
            def _():
                fire_idx(1, t0 + 3)

            acc_out(1, t0 + 1)

    return pool_kernel(x3, table)


BB = 512


def _mlp_tc(sparse, gamma, beta, W1, b1, W2, b2, W3, b3, nb=B):
    def body(p_ref, g_ref, be_ref, w1_ref, b1_ref, w2_ref, b2_ref, w3_ref,
             b3_ref, o_ref):
        sp = p_ref[...]
        mu = jnp.mean(sp, axis=-1, keepdims=True)
        var = jnp.mean((sp - mu) ** 2, axis=-1, keepdims=True)
        h = (sp - mu) / jnp.sqrt(var + 1e-5) * g_ref[...] + be_ref[...]
        h = jnp.maximum(
            jnp.dot(h.astype(jnp.bfloat16), w1_ref[...].astype(jnp.bfloat16),
                    preferred_element_type=jnp.float32)
            + b1_ref[...], 0.0)
        h = jnp.maximum(
            jnp.dot(h.astype(jnp.bfloat16), w2_ref[...].astype(jnp.bfloat16),
                    preferred_element_type=jnp.float32)
            + b2_ref[...], 0.0)
        o = jnp.dot(h, w3_ref[...], preferred_element_type=jnp.float32) + b3_ref[...]
        o_ref[...] = jax.nn.sigmoid(o)

    return pl.pallas_call(
        body,
        grid=(nb // BB,),
        in_specs=[
            pl.BlockSpec((BB, H), lambda i: (i, 0)),
            pl.BlockSpec((H,), lambda i: (0,)),
            pl.BlockSpec((H,), lambda i: (0,)),
            pl.BlockSpec((H, 1024), lambda i: (0, 0)),
            pl.BlockSpec((1024,), lambda i: (0,)),
            pl.BlockSpec((1024, 512), lambda i: (0, 0)),
            pl.BlockSpec((512,), lambda i: (0,)),
            pl.BlockSpec((512, 1), lambda i: (0, 0)),
            pl.BlockSpec((1,), lambda i: (0,)),
        ],
        out_specs=pl.BlockSpec((BB, 1), lambda i: (i, 0)),
        out_shape=jax.ShapeDtypeStruct((nb, 1), jnp.float32),
    )(sparse, gamma, beta, W1, b1, W2, b2, W3, b3)


def kernel(x, table, gamma, beta, W1, b1, W2, b2, W3, b3):
    x3 = x.transpose(1, 2, 0)
    tbl_t = table.T
    tail = jnp.pad(tbl_t[:, NFULL * VB:], ((0, 0), (0, 640 - VTAIL)))
    tbl_lin = _transpose_sc(tbl_t, tail).reshape(V, D)
    outs = []
    ncb = NBLK // 2
    for c in range(2):
        pooled = _pool_sc(x3, tbl_lin, ncb, c * ncb)
        sparse = pooled.reshape(ncb * BBLK, H)
        outs.append(_mlp_tc(sparse, gamma, beta, W1, b1, W2, b2, W3, b3,
                            nb=ncb * BBLK))
    return jnp.concatenate(outs, axis=0)
